# Initial kernel scaffold; baseline (speedup 1.0000x reference)
import jax
import jax.numpy as jnp
from jax import lax
from jax.experimental import pallas as pl
from jax.experimental.pallas import tpu as pltpu
from jax.experimental.pallas import tpu_sc as plsc

N = 10000
E = 320000
F_IN = 128
H = 128
G = 64

NC = 2
NS = 16
NW = NC * NS
CHUNK = 128
CH = (E + NW * CHUNK - 1) // (NW * CHUNK)
EP = NW * CH * CHUNK
NP = 10240
RPT = NP // NS
BLK = 256
NBLK = NP // BLK

_mesh = plsc.VectorSubcoreMesh(core_axis_name="c", subcore_axis_name="s")


def _sc_degree(dst_p):
    def body(dst_hbm, out_hbm, dstb, buf, acc, sem):
        cid = lax.axis_index("c")
        sid = lax.axis_index("s")
        wid = sid * NC + cid

        pltpu.sync_copy(dst_hbm.at[wid], dstb)

        zero = jnp.zeros((16,), jnp.float32)

        @pl.loop(0, CHUNK)
        def _z(r):
            buf[r, :] = zero

        @pl.loop(0, RPT // CHUNK)
        def _zi(i):
            pltpu.sync_copy(buf, acc.at[pl.ds(sid * RPT + i * CHUNK, CHUNK)])

        one = jnp.ones((16,), jnp.float32)

        @pl.loop(0, CHUNK)
        def _o(r):
            buf[r, :] = one

        plsc.subcore_barrier()

        @pl.loop(0, CH)
        def _s(j):
            pltpu.sync_copy(buf, acc.at[dstb.at[j]], add=True)

        plsc.subcore_barrier()

        pltpu.sync_copy(
            acc.at[pl.ds(sid * RPT, RPT)],
            out_hbm.at[cid].at[pl.ds(sid * RPT, RPT)],
        )

    kern = pl.kernel(
        body,
        out_type=jax.ShapeDtypeStruct((NC, NP, 16), jnp.float32),
        mesh=_mesh,
        scratch_types=[
            pltpu.VMEM((CH, CHUNK), jnp.int32),
            pltpu.VMEM((CHUNK, 16), jnp.float32),
            pltpu.VMEM_SHARED((NP, 16), jnp.float32),
            pltpu.SemaphoreType.DMA,
        ],
    )
    return kern(dst_p)


def _sc_scatter(hs, src_p, dst_p):
    def body(hs_hbm, src_hbm, dst_hbm, out_hbm, srcb, dstb, gbuf, acc, sem):
        cid = lax.axis_index("c")
        sid = lax.axis_index("s")
        wid = sid * NC + cid

        pltpu.sync_copy(src_hbm.at[wid], srcb)
        pltpu.sync_copy(dst_hbm.at[wid], dstb)

        zero = jnp.zeros((16,), jnp.float32)

        @pl.loop(0, CHUNK)
        def _z(r):
            @pl.loop(0, H // 16)
            def _zc(c):
                gbuf[r, pl.ds(c * 16, 16)] = zero

        @pl.loop(0, RPT // CHUNK)
        def _zi(i):
            pltpu.sync_copy(gbuf, acc.at[pl.ds(sid * RPT + i * CHUNK, CHUNK)])

        plsc.subcore_barrier()

        @pl.loop(0, CH)
        def _s(j):
            pltpu.async_copy(hs_hbm.at[srcb.at[j]], gbuf, sem).wait()
            pltpu.sync_copy(gbuf, acc.at[dstb.at[j]], add=True)

        plsc.subcore_barrier()

        pltpu.sync_copy(
            acc.at[pl.ds(sid * RPT, RPT)],
            out_hbm.at[cid].at[pl.ds(sid * RPT, RPT)],
        )

    kern = pl.kernel(
        body,
        out_type=jax.ShapeDtypeStruct((NC, NP, 128), jnp.float32),
        mesh=_mesh,
        scratch_types=[
            pltpu.VMEM((CH, CHUNK), jnp.int32),
            pltpu.VMEM((CH, CHUNK), jnp.int32),
            pltpu.VMEM((CHUNK, 128), jnp.float32),
            pltpu.VMEM_SHARED((NP, 128), jnp.float32),
            pltpu.SemaphoreType.DMA,
        ],
    )
    return kern(hs, src_p, dst_p)


def _d_block(c0, c1):
    deg = c0[:, 0:1] + c1[:, 0:1] + 1.0
    return lax.rsqrt(deg)


def _tc1_body(x_ref, w_ref, c0_ref, c1_ref, o_ref):
    d = _d_block(c0_ref[...], c1_ref[...])
    hw = jnp.dot(x_ref[...], w_ref[...], preferred_element_type=jnp.float32)
    o_ref[...] = d * hw


def _tc1(x_p, W1, cnt0, cnt1):
    return pl.pallas_call(
        _tc1_body,
        grid=(NBLK,),
        in_specs=[
            pl.BlockSpec((BLK, F_IN), lambda k: (k, 0)),
            pl.BlockSpec((F_IN, H), lambda k: (0, 0)),
            pl.BlockSpec((BLK, 16), lambda k: (k, 0)),
            pl.BlockSpec((BLK, 16), lambda k: (k, 0)),
        ],
        out_specs=pl.BlockSpec((BLK, H), lambda k: (k, 0)),
        out_shape=jax.ShapeDtypeStruct((NP, H), jnp.float32),
    )(x_p, W1, cnt0, cnt1)


def _tc2_body(p0_ref, p1_ref, hs_ref, c0_ref, c1_ref, b_ref, w_ref, o_ref):
    d = _d_block(c0_ref[...], c1_ref[...])
    h = jax.nn.relu(d * (p0_ref[0] + p1_ref[0] + hs_ref[...])
                    + b_ref[0:1, :])
    hw = jnp.dot(h, w_ref[...], preferred_element_type=jnp.float32)
    o_ref[...] = d * hw


def _tc2(p, hs1, cnt0, cnt1, b1r, W2):
    return pl.pallas_call(
        _tc2_body,
        grid=(NBLK,),
        in_specs=[
            pl.BlockSpec((1, BLK, H), lambda k: (0, k, 0)),
            pl.BlockSpec((1, BLK, H), lambda k: (1, k, 0)),
            pl.BlockSpec((BLK, H), lambda k: (k, 0)),
            pl.BlockSpec((BLK, 16), lambda k: (k, 0)),
            pl.BlockSpec((BLK, 16), lambda k: (k, 0)),
            pl.BlockSpec((8, H), lambda k: (0, 0)),
            pl.BlockSpec((H, H), lambda k: (0, 0)),
        ],
        out_specs=pl.BlockSpec((BLK, H), lambda k: (k, 0)),
        out_shape=jax.ShapeDtypeStruct((NP, H), jnp.float32),
    )(p, p, hs1, cnt0, cnt1, b1r, W2)


def _tc3_body(p0_ref, p1_ref, hs_ref, c0_ref, c1_ref, b_ref, bt_ref,
              wf1_ref, bf1_ref, wf2_ref, bf2_ref, o_ref, pool_ref):
    k = pl.program_id(0)

    @pl.when(k == 0)
    def _():
        pool_ref[...] = jnp.zeros_like(pool_ref)

    d = _d_block(c0_ref[...], c1_ref[...])
    h2 = jax.nn.relu(d * (p0_ref[0] + p1_ref[0] + hs_ref[...])
                     + b_ref[0:1, :])
    bt = bt_ref[0]
    onehot = (lax.broadcasted_iota(jnp.int32, (G, BLK), 0)
              == jnp.broadcast_to(bt, (G, BLK))).astype(jnp.float32)
    pool_ref[...] += jnp.dot(onehot, h2, preferred_element_type=jnp.float32)

    @pl.when(k == NBLK - 1)
    def _():
        r = jax.nn.relu(
            jnp.dot(pool_ref[...], wf1_ref[...],
                    preferred_element_type=jnp.float32)
            + bf1_ref[0:1, :])
        o_ref[...] = (jnp.dot(r, wf2_ref[...],
                              preferred_element_type=jnp.float32)
                      + bf2_ref[0:1, :])


def _tc3(p, hs2, cnt0, cnt1, b2r, batch3, Wfc1, bfc1r, Wfc2p, bfc2r):
    return pl.pallas_call(
        _tc3_body,
        grid=(NBLK,),
        in_specs=[
            pl.BlockSpec((1, BLK, H), lambda k: (0, k, 0)),
            pl.BlockSpec((1, BLK, H), lambda k: (1, k, 0)),
            pl.BlockSpec((BLK, H), lambda k: (k, 0)),
            pl.BlockSpec((BLK, 16), lambda k: (k, 0)),
            pl.BlockSpec((BLK, 16), lambda k: (k, 0)),
            pl.BlockSpec((8, H), lambda k: (0, 0)),
            pl.BlockSpec((1, 1, BLK), lambda k: (k, 0, 0)),
            pl.BlockSpec((H, H), lambda k: (0, 0)),
            pl.BlockSpec((8, H), lambda k: (0, 0)),
            pl.BlockSpec((H, H), lambda k: (0, 0)),
            pl.BlockSpec((8, H), lambda k: (0, 0)),
        ],
        out_specs=pl.BlockSpec((G, H), lambda k: (0, 0)),
        out_shape=jax.ShapeDtypeStruct((G, H), jnp.float32),
        scratch_shapes=[pltpu.VMEM((G, H), jnp.float32)],
    )(p, p, hs2, cnt0, cnt1, b2r, batch3, Wfc1, bfc1r, Wfc2p, bfc2r)


@jax.jit
def kernel(x, edge_index, batch, W1, b1, W2, b2, Wfc1, bfc1, Wfc2, bfc2):
    src = edge_index[0]
    dst = edge_index[1]
    pad_e = jnp.full((EP - E,), N, jnp.int32)
    src_p = jnp.concatenate([src, pad_e]).reshape(NW, CH, CHUNK)
    dst_p = jnp.concatenate([dst, pad_e]).reshape(NW, CH, CHUNK)

    x_p = jnp.pad(x, ((0, NP - N), (0, 0)))
    batch3 = jnp.concatenate(
        [batch, jnp.full((NP - N,), G, jnp.int32)]).reshape(NBLK, 1, BLK)

    b1r = jnp.broadcast_to(b1[None, :], (8, H))
    b2r = jnp.broadcast_to(b2[None, :], (8, H))
    bfc1r = jnp.broadcast_to(bfc1[None, :], (8, H))
    bfc2r = jnp.broadcast_to(bfc2.reshape(1, 1), (8, H))
    Wfc2p = jnp.pad(Wfc2, ((0, 0), (0, H - 1)))

    cnt = _sc_degree(dst_p)
    cnt0, cnt1 = cnt[0], cnt[1]

    hs1 = _tc1(x_p, W1, cnt0, cnt1)
    p1 = _sc_scatter(hs1, src_p, dst_p)
    hs2 = _tc2(p1, hs1, cnt0, cnt1, b1r, W2)
    p2 = _sc_scatter(hs2, src_p, dst_p)
    outG = _tc3(p2, hs2, cnt0, cnt1, b2r, batch3,
                Wfc1, bfc1r, Wfc2p, bfc2r)
    return outG[:, :1]


# speedup vs baseline: 12.9283x; 12.9283x over previous
import dataclasses

import jax
import jax.numpy as jnp
from jax import lax
from jax.experimental import pallas as pl
from jax.experimental.pallas import tpu as pltpu
from jax.experimental.pallas import tpu_sc as plsc

N = 10000
E = 320000
F_IN = 128
H = 128
G = 64

NC = 2
NS = 16
NW = NC * NS
CHUNK = 128
CH = (E + NW * CHUNK - 1) // (NW * CHUNK)
EP = NW * CH * CHUNK
NP = 10240
RPT = NP // NS
BLK = 256
NBLK = NP // BLK

def _mesh():
    return plsc.VectorSubcoreMesh(
        core_axis_name="c", subcore_axis_name="s",
        num_cores=NC, num_subcores=NS)


def _sc_degree(dst_flat):
    def body(dst_hbm, out_hbm, dstb, accd, sem):
        cid = lax.axis_index("c")
        sid = lax.axis_index("s")
        wid = sid * NC + cid

        pltpu.sync_copy(dst_hbm.at[wid], dstb)

        zero = jnp.zeros((16,), jnp.float32)

        @pl.loop(0, NP // 16)
        def _z(i):
            accd[pl.ds(i * 16, 16)] = zero

        ones16 = jnp.ones((16,), jnp.float32)

        @pl.loop(0, CH * CHUNK // 16)
        def _s(i):
            idx = dstb[pl.ds(i * 16, 16)]
            plsc.addupdate_scatter(accd, [idx], ones16)

        pltpu.sync_copy(accd, out_hbm.at[wid])

    cp = pltpu.CompilerParams()
    if "needs_layout_passes" in pltpu.CompilerParams.__dataclass_fields__:
        cp = dataclasses.replace(cp, needs_layout_passes=False)
    kern = pl.kernel(
        body,
        out_type=jax.ShapeDtypeStruct((NW, NP), jnp.float32),
        mesh=_mesh(),
        compiler_params=cp,
        scratch_types=[
            pltpu.VMEM((CH * CHUNK,), jnp.int32),
            pltpu.VMEM((NP,), jnp.float32),
            pltpu.SemaphoreType.DMA,
        ],
    )
    return kern(dst_flat)


def _sc_scatter(hs, src_p, dst_p):
    def body(hs_hbm, src_hbm, dst_hbm, out_hbm, srcb, dstb, gbuf, acc, sem):
        cid = lax.axis_index("c")
        sid = lax.axis_index("s")
        wid = sid * NC + cid

        pltpu.sync_copy(src_hbm.at[wid], srcb)
        pltpu.sync_copy(dst_hbm.at[wid], dstb)

        zero = jnp.zeros((16,), jnp.float32)

        @pl.loop(0, CHUNK)
        def _z(r):
            @pl.loop(0, H // 16)
            def _zc(c):
                gbuf[r, pl.ds(c * 16, 16)] = zero

        @pl.loop(0, RPT // CHUNK)
        def _zi(i):
            pltpu.sync_copy(gbuf, acc.at[pl.ds(sid * RPT + i * CHUNK, CHUNK)])

        plsc.subcore_barrier()

        @pl.loop(0, CH)
        def _s(j):
            pltpu.async_copy(hs_hbm.at[srcb.at[j]], gbuf, sem).wait()
            pltpu.sync_copy(gbuf, acc.at[dstb.at[j]], add=True)

        plsc.subcore_barrier()

        pltpu.sync_copy(
            acc.at[pl.ds(sid * RPT, RPT)],
            out_hbm.at[cid].at[pl.ds(sid * RPT, RPT)],
        )

    kern = pl.kernel(
        body,
        out_type=jax.ShapeDtypeStruct((NC, NP, 128), jnp.float32),
        mesh=_mesh(),
        scratch_types=[
            pltpu.VMEM((CH, CHUNK), jnp.int32),
            pltpu.VMEM((CH, CHUNK), jnp.int32),
            pltpu.VMEM((CHUNK, 128), jnp.float32),
            pltpu.VMEM_SHARED((NP, 128), jnp.float32),
            pltpu.SemaphoreType.DMA,
        ],
    )
    return kern(hs, src_p, dst_p)


def _d_block(cnt):
    ones = jnp.ones((NW, 1), jnp.float32)
    deg = lax.dot_general(cnt, ones, (((0,), (0,)), ((), ())),
                          preferred_element_type=jnp.float32) + 1.0
    return lax.rsqrt(deg)


def _tc1_body(x_ref, w_ref, c_ref, o_ref):
    d = _d_block(c_ref[...])
    hw = jnp.dot(x_ref[...], w_ref[...], preferred_element_type=jnp.float32)
    o_ref[...] = d * hw


def _tc1(x_p, W1, cnt):
    return pl.pallas_call(
        _tc1_body,
        grid=(NBLK,),
        in_specs=[
            pl.BlockSpec((BLK, F_IN), lambda k: (k, 0)),
            pl.BlockSpec((F_IN, H), lambda k: (0, 0)),
            pl.BlockSpec((NW, BLK), lambda k: (0, k)),
        ],
        out_specs=pl.BlockSpec((BLK, H), lambda k: (k, 0)),
        out_shape=jax.ShapeDtypeStruct((NP, H), jnp.float32),
    )(x_p, W1, cnt)


def _tc2_body(p0_ref, p1_ref, hs_ref, c_ref, b_ref, w_ref, o_ref):
    d = _d_block(c_ref[...])
    h = jax.nn.relu(d * (p0_ref[0] + p1_ref[0] + hs_ref[...])
                    + b_ref[0:1, :])
    hw = jnp.dot(h, w_ref[...], preferred_element_type=jnp.float32)
    o_ref[...] = d * hw


def _tc2(p, hs1, cnt, b1r, W2):
    return pl.pallas_call(
        _tc2_body,
        grid=(NBLK,),
        in_specs=[
            pl.BlockSpec((1, BLK, H), lambda k: (0, k, 0)),
            pl.BlockSpec((1, BLK, H), lambda k: (1, k, 0)),
            pl.BlockSpec((BLK, H), lambda k: (k, 0)),
            pl.BlockSpec((NW, BLK), lambda k: (0, k)),
            pl.BlockSpec((8, H), lambda k: (0, 0)),
            pl.BlockSpec((H, H), lambda k: (0, 0)),
        ],
        out_specs=pl.BlockSpec((BLK, H), lambda k: (k, 0)),
        out_shape=jax.ShapeDtypeStruct((NP, H), jnp.float32),
    )(p, p, hs1, cnt, b1r, W2)


def _tc3_body(p0_ref, p1_ref, hs_ref, c_ref, b_ref, bt_ref,
              wf1_ref, bf1_ref, wf2_ref, bf2_ref, o_ref, pool_ref):
    k = pl.program_id(0)

    @pl.when(k == 0)
    def _():
        pool_ref[...] = jnp.zeros_like(pool_ref)

    d = _d_block(c_ref[...])
    h2 = jax.nn.relu(d * (p0_ref[0] + p1_ref[0] + hs_ref[...])
                     + b_ref[0:1, :])
    bt = bt_ref[0]
    onehot = (lax.broadcasted_iota(jnp.int32, (G, BLK), 0)
              == jnp.broadcast_to(bt, (G, BLK))).astype(jnp.float32)
    pool_ref[...] += jnp.dot(onehot, h2, preferred_element_type=jnp.float32)

    @pl.when(k == NBLK - 1)
    def _():
        r = jax.nn.relu(
            jnp.dot(pool_ref[...], wf1_ref[...],
                    preferred_element_type=jnp.float32)
            + bf1_ref[0:1, :])
        o_ref[...] = (jnp.dot(r, wf2_ref[...],
                              preferred_element_type=jnp.float32)
                      + bf2_ref[0:1, :])


def _tc3(p, hs2, cnt, b2r, batch3, Wfc1, bfc1r, Wfc2p, bfc2r):
    return pl.pallas_call(
        _tc3_body,
        grid=(NBLK,),
        in_specs=[
            pl.BlockSpec((1, BLK, H), lambda k: (0, k, 0)),
            pl.BlockSpec((1, BLK, H), lambda k: (1, k, 0)),
            pl.BlockSpec((BLK, H), lambda k: (k, 0)),
            pl.BlockSpec((NW, BLK), lambda k: (0, k)),
            pl.BlockSpec((8, H), lambda k: (0, 0)),
            pl.BlockSpec((1, 1, BLK), lambda k: (k, 0, 0)),
            pl.BlockSpec((H, H), lambda k: (0, 0)),
            pl.BlockSpec((8, H), lambda k: (0, 0)),
            pl.BlockSpec((H, H), lambda k: (0, 0)),
            pl.BlockSpec((8, H), lambda k: (0, 0)),
        ],
        out_specs=pl.BlockSpec((G, H), lambda k: (0, 0)),
        out_shape=jax.ShapeDtypeStruct((G, H), jnp.float32),
        scratch_shapes=[pltpu.VMEM((G, H), jnp.float32)],
    )(p, p, hs2, cnt, b2r, batch3, Wfc1, bfc1r, Wfc2p, bfc2r)


@jax.jit
def kernel(x, edge_index, batch, W1, b1, W2, b2, Wfc1, bfc1, Wfc2, bfc2):
    src = edge_index[0]
    dst = edge_index[1]
    pad_e = jnp.full((EP - E,), N, jnp.int32)
    src_p = jnp.concatenate([src, pad_e]).reshape(NW, CH, CHUNK)
    dst_p = jnp.concatenate([dst, pad_e]).reshape(NW, CH, CHUNK)

    x_p = jnp.pad(x, ((0, NP - N), (0, 0)))
    batch3 = jnp.concatenate(
        [batch, jnp.full((NP - N,), G, jnp.int32)]).reshape(NBLK, 1, BLK)

    b1r = jnp.broadcast_to(b1[None, :], (8, H))
    b2r = jnp.broadcast_to(b2[None, :], (8, H))
    bfc1r = jnp.broadcast_to(bfc1[None, :], (8, H))
    bfc2r = jnp.broadcast_to(bfc2.reshape(1, 1), (8, H))
    Wfc2p = jnp.pad(Wfc2, ((0, 0), (0, H - 1)))

    cnt = _sc_degree(dst_p.reshape(NW, CH * CHUNK))

    hs1 = _tc1(x_p, W1, cnt)
    p1 = _sc_scatter(hs1, src_p, dst_p)
    hs2 = _tc2(p1, hs1, cnt, b1r, W2)
    p2 = _sc_scatter(hs2, src_p, dst_p)
    outG = _tc3(p2, hs2, cnt, b2r, batch3,
                Wfc1, bfc1r, Wfc2p, bfc2r)
    return outG[:, :1]
